# baseline (speedup 1.0000x reference)
import jax
import jax.numpy as jnp
from jax import lax
from jax.experimental import pallas as pl
from jax.experimental.pallas import tpu as pltpu
from jax.experimental.pallas import tpu_sc as plsc

N = 10000
E = 320000
D = 128
NEG_SLOPE = 0.2

NP = 10240
EP = 327680
BLK = 1024
GRID = NP // BLK
NCHUNK = 80
CH = 128
EROWS = NCHUNK
SP_ROWS = NP // 16


def _lrelu(t):
    return jnp.maximum(t, NEG_SLOPE * t)


_GDN = jax.lax.GatherDimensionNumbers(
    offset_dims=(), collapsed_slice_dims=(0,), start_index_map=(0,))


def _splat_lane(v16, k):
    idx = jnp.full((16,), k, jnp.int32)
    return jax.lax.gather(
        v16, idx[:, None], _GDN, (1,),
        mode=jax.lax.GatherScatterMode.PROMISE_IN_BOUNDS)


def _attn_epilogue(h_blk, asv, adv, asP_ref, adP_ref, am_ref, smax_ref, i):
    as_b = jnp.sum(h_blk * asv, axis=1)
    ad_b = jnp.sum(h_blk * adv, axis=1)
    asP_ref[...] = as_b.reshape(8, 128)
    adP_ref[...] = ad_b.reshape(8, 128)
    m = jnp.max(as_b)

    @pl.when(i == 0)
    def _():
        smax_ref[0, 0] = m

    @pl.when(i > 0)
    def _():
        smax_ref[0, 0] = jnp.maximum(smax_ref[0, 0], m)

    @pl.when(i == GRID - 1)
    def _():
        am_ref[...] = jnp.full((8, 128), smax_ref[0, 0], jnp.float32)


def _tc_pre_body(x_ref, w_ref, asv_ref, adv_ref,
                 h_ref, asP_ref, adP_ref, am_ref, smax_ref):
    i = pl.program_id(0)
    h_blk = jnp.dot(x_ref[...], w_ref[...], preferred_element_type=jnp.float32)
    h_ref[...] = h_blk
    _attn_epilogue(h_blk, asv_ref[...], adv_ref[...], asP_ref, adP_ref,
                   am_ref, smax_ref, i)


def _tc_pre(x, w, asv, adv):
    return pl.pallas_call(
        _tc_pre_body,
        grid=(GRID,),
        in_specs=[
            pl.BlockSpec((BLK, D), lambda i: (i, 0)),
            pl.BlockSpec((D, D), lambda i: (0, 0)),
            pl.BlockSpec((1, D), lambda i: (0, 0)),
            pl.BlockSpec((1, D), lambda i: (0, 0)),
        ],
        out_specs=[
            pl.BlockSpec((BLK, D), lambda i: (i, 0)),
            pl.BlockSpec((8, 128), lambda i: (i, 0)),
            pl.BlockSpec((8, 128), lambda i: (i, 0)),
            pl.BlockSpec((8, 128), lambda i: (0, 0)),
        ],
        out_shape=[
            jax.ShapeDtypeStruct((NP, D), jnp.float32),
            jax.ShapeDtypeStruct((GRID * 8, 128), jnp.float32),
            jax.ShapeDtypeStruct((GRID * 8, 128), jnp.float32),
            jax.ShapeDtypeStruct((8, 128), jnp.float32),
        ],
        scratch_shapes=[pltpu.SMEM((1, 1), jnp.float32)],
    )(x, w, asv, adv)


def _combine(accA_ref, accB_ref, denA_ref, denB_ref, h_ref, asP_ref, adP_ref,
             am_ref, b_ref):
    as_b = asP_ref[...].reshape(BLK)
    ad_b = adP_ref[...].reshape(BLK)
    am = am_ref[0, 0]
    exs = jnp.exp(_lrelu(as_b + ad_b) - _lrelu(am + ad_b))
    num = accA_ref[...] + accB_ref[...] + exs[:, None] * h_ref[...]
    den = denA_ref[...].reshape(BLK) + denB_ref[...].reshape(BLK) + exs
    return num / (den + 1e-16)[:, None] + b_ref[...]


_SPEC_ROWS = pl.BlockSpec((BLK, D), lambda i: (i, 0))
_SPEC_8x128 = pl.BlockSpec((8, 128), lambda i: (i, 0))
_SPEC_CONST = pl.BlockSpec((8, 128), lambda i: (0, 0))
_SPEC_VEC = pl.BlockSpec((1, D), lambda i: (0, 0))


def _tc_mid_body(accA_ref, accB_ref, denA_ref, denB_ref, h_ref, asP_ref,
                 adP_ref, am_ref, b_ref, w_ref, asv_ref, adv_ref,
                 h2_ref, asP2_ref, adP2_ref, am2_ref, smax_ref):
    i = pl.program_id(0)
    feats = _combine(accA_ref, accB_ref, denA_ref, denB_ref, h_ref, asP_ref,
                     adP_ref, am_ref, b_ref)
    h2_blk = jnp.dot(feats, w_ref[...], preferred_element_type=jnp.float32)
    h2_ref[...] = h2_blk
    _attn_epilogue(h2_blk, asv_ref[...], adv_ref[...], asP2_ref, adP2_ref,
                   am2_ref, smax_ref, i)


def _tc_mid(accA, accB, denA, denB, h, asP, adP, am, b, w, asv, adv):
    return pl.pallas_call(
        _tc_mid_body,
        grid=(GRID,),
        in_specs=[
            _SPEC_ROWS, _SPEC_ROWS, _SPEC_8x128, _SPEC_8x128, _SPEC_ROWS,
            _SPEC_8x128, _SPEC_8x128, _SPEC_CONST, _SPEC_VEC,
            pl.BlockSpec((D, D), lambda i: (0, 0)), _SPEC_VEC, _SPEC_VEC,
        ],
        out_specs=[
            _SPEC_ROWS, _SPEC_8x128, _SPEC_8x128, _SPEC_CONST,
        ],
        out_shape=[
            jax.ShapeDtypeStruct((NP, D), jnp.float32),
            jax.ShapeDtypeStruct((GRID * 8, 128), jnp.float32),
            jax.ShapeDtypeStruct((GRID * 8, 128), jnp.float32),
            jax.ShapeDtypeStruct((8, 128), jnp.float32),
        ],
        scratch_shapes=[pltpu.SMEM((1, 1), jnp.float32)],
    )(accA, accB, denA, denB, h, asP, adP, am, b, w, asv, adv)


def _tc_post_body(accA_ref, accB_ref, denA_ref, denB_ref, h_ref, asP_ref,
                  adP_ref, am_ref, b_ref, out_ref):
    out_ref[...] = _combine(accA_ref, accB_ref, denA_ref, denB_ref, h_ref,
                            asP_ref, adP_ref, am_ref, b_ref)


def _tc_post(accA, accB, denA, denB, h, asP, adP, am, b):
    return pl.pallas_call(
        _tc_post_body,
        grid=(GRID,),
        in_specs=[
            _SPEC_ROWS, _SPEC_ROWS, _SPEC_8x128, _SPEC_8x128, _SPEC_ROWS,
            _SPEC_8x128, _SPEC_8x128, _SPEC_CONST, _SPEC_VEC,
        ],
        out_specs=pl.BlockSpec((BLK, D), lambda i: (i, 0)),
        out_shape=jax.ShapeDtypeStruct((N, D), jnp.float32),
    )(accA, accB, denA, denB, h, asP, adP, am, b)


NSUP = NCHUNK // 8


def _sc_alpha_body(asP_hbm, adP_hbm, am_hbm, src_hbm, dst_hbm, z1_hbm,
                   ex_hbm, denA_hbm, denB_hbm,
                   as_v, ad_v, am_v, src8_v, dst8_v, ex8_v, dsem,
                   den_sp):
    cid = lax.axis_index("c")
    sid = lax.axis_index("s")
    w = cid * 16 + sid

    pltpu.sync_copy(asP_hbm, as_v)
    pltpu.sync_copy(adP_hbm, ad_v)
    pltpu.sync_copy(am_hbm.at[pl.ds(0, 1)], am_v)

    @pl.when(sid == 0)
    def _():
        pltpu.sync_copy(z1_hbm, den_sp)

    plsc.subcore_barrier()

    amax16 = am_v[0, pl.ds(0, 16)]

    def superstep(g, carry):
        br = pl.ds(w * EROWS + g * 8, 8)
        pltpu.sync_copy(src_hbm.at[br], src8_v)
        pltpu.sync_copy(dst_hbm.at[br], dst8_v)

        waits = []
        for c8 in range(8):
            for j in range(8):
                sl = pl.ds(16 * j, 16)
                s16 = src8_v[c8, sl]
                d16 = dst8_v[c8, sl]
                a_s = plsc.load_gather(as_v, [s16 >> 7, s16 & 127])
                a_d = plsc.load_gather(ad_v, [d16 >> 7, d16 & 127])
                al = _lrelu(a_s + a_d)
                ub = _lrelu(amax16 + a_d)
                ex8_v[c8, sl] = jnp.exp(al - ub)
            waits.append(pltpu.async_copy(
                ex8_v.at[c8], den_sp.at[dst8_v.at[c8]], dsem, add=True))
        pltpu.sync_copy(ex8_v, ex_hbm.at[br])
        for h in waits:
            h.wait()
        return carry

    lax.fori_loop(0, NSUP, superstep, 0)
    plsc.subcore_barrier()

    @pl.when(jnp.logical_and(cid == 0, sid == 0))
    def _():
        pltpu.sync_copy(den_sp, denA_hbm)

    @pl.when(jnp.logical_and(cid == 1, sid == 0))
    def _():
        pltpu.sync_copy(den_sp, denB_hbm)


def _sc_rows_body(h_hbm, ex_hbm, src_hbm, dst_hbm, z128_hbm,
                  accA_hbm, accB_hbm,
                  src8_v, dst8_v, ex8_v, rows_a, rows_b, sem_a, sem_b,
                  ssem_a, ssem_b, out_sp):
    cid = lax.axis_index("c")
    sid = lax.axis_index("s")
    w = cid * 16 + sid

    rz = pl.ds(sid * SP_ROWS, SP_ROWS)
    pltpu.sync_copy(z128_hbm.at[rz], out_sp.at[rz])
    plsc.subcore_barrier()

    def stage(batch, half):
        br = pl.ds(w * EROWS + batch * 8, 8)
        hf = pl.ds(half * 8, 8)
        pltpu.sync_copy(src_hbm.at[br], src8_v.at[hf])
        pltpu.sync_copy(dst_hbm.at[br], dst8_v.at[hf])
        pltpu.sync_copy(ex_hbm.at[br], ex8_v.at[hf])

    stage(0, 0)
    pltpu.async_copy(h_hbm.at[src8_v.at[0]], rows_a, sem_a)

    def superstep(g, carry):
        gp = g & 1

        @pl.when(g < NSUP - 1)
        def _():
            stage(g + 1, 1 - gp)

        for c8 in range(8):
            rows, sem = (rows_a, sem_a) if c8 % 2 == 0 else (rows_b, sem_b)
            orows, osem = (rows_b, sem_b) if c8 % 2 == 0 else (rows_a, sem_a)
            ossem = ssem_b if c8 % 2 == 0 else ssem_a
            myssem = ssem_a if c8 % 2 == 0 else ssem_b
            r = gp * 8 + c8

            pltpu.make_async_copy(
                h_hbm.at[src8_v.at[r]], rows, sem).wait()

            def _wait_prev():
                pltpu.make_async_copy(
                    orows, out_sp.at[dst8_v.at[r]], ossem).wait()

            if c8 == 0:
                @pl.when(g > 0)
                def _():
                    _wait_prev()
            else:
                _wait_prev()

            if c8 < 7:
                pltpu.async_copy(h_hbm.at[src8_v.at[r + 1]], orows, osem)
            else:
                @pl.when(g < NSUP - 1)
                def _():
                    pltpu.async_copy(
                        h_hbm.at[src8_v.at[(1 - gp) * 8]], orows, osem)

            for grp in range(8):
                e16 = ex8_v[r, pl.ds(16 * grp, 16)]

                @plsc.parallel_loop(0, 16, unroll=4)
                def _(k2, e16=e16, grp=grp, rows=rows):
                    e = _splat_lane(e16, k2)
                    for j in range(8):
                        sl = pl.ds(16 * j, 16)
                        rows[16 * grp + k2, sl] = rows[16 * grp + k2, sl] * e

            pltpu.async_copy(rows, out_sp.at[dst8_v.at[r]], myssem, add=True)
        return carry

    lax.fori_loop(0, NSUP, superstep, 0)
    pltpu.make_async_copy(
        rows_b, out_sp.at[dst8_v.at[15]], ssem_b).wait()
    plsc.subcore_barrier()

    @pl.when(cid == 0)
    def _():
        pltpu.sync_copy(out_sp.at[rz], accA_hbm.at[rz])

    @pl.when(cid == 1)
    def _():
        pltpu.sync_copy(out_sp.at[rz], accB_hbm.at[rz])


def _sc_edge(h, asP, adP, am, srcP, dstP, z128, z1):
    mesh = plsc.VectorSubcoreMesh(core_axis_name="c", subcore_axis_name="s")
    alpha = pl.kernel(
        _sc_alpha_body,
        mesh=mesh,
        compiler_params=pltpu.CompilerParams(needs_layout_passes=False),
        out_type=[
            jax.ShapeDtypeStruct((EP // CH, CH), jnp.float32),
            jax.ShapeDtypeStruct((NP,), jnp.float32),
            jax.ShapeDtypeStruct((NP,), jnp.float32),
        ],
        scratch_types=[
            pltpu.VMEM((GRID * 8, 128), jnp.float32),
            pltpu.VMEM((GRID * 8, 128), jnp.float32),
            pltpu.VMEM((1, 128), jnp.float32),
            pltpu.VMEM((8, CH), jnp.int32),
            pltpu.VMEM((8, CH), jnp.int32),
            pltpu.VMEM((8, CH), jnp.float32),
            pltpu.SemaphoreType.DMA,
            pltpu.VMEM_SHARED((NP,), jnp.float32),
        ],
    )
    exP, denA, denB = alpha(asP, adP, am, srcP, dstP, z1)

    rows = pl.kernel(
        _sc_rows_body,
        mesh=mesh,
        compiler_params=pltpu.CompilerParams(needs_layout_passes=False),
        out_type=[
            jax.ShapeDtypeStruct((NP, D), jnp.float32),
            jax.ShapeDtypeStruct((NP, D), jnp.float32),
        ],
        scratch_types=[
            pltpu.VMEM((16, CH), jnp.int32),
            pltpu.VMEM((16, CH), jnp.int32),
            pltpu.VMEM((16, CH), jnp.float32),
            pltpu.VMEM((CH, D), jnp.float32),
            pltpu.VMEM((CH, D), jnp.float32),
            pltpu.SemaphoreType.DMA,
            pltpu.SemaphoreType.DMA,
            pltpu.SemaphoreType.DMA,
            pltpu.SemaphoreType.DMA,
            pltpu.VMEM_SHARED((NP, D), jnp.float32),
        ],
    )
    accA, accB = rows(h, exP, srcP, dstP, z128)
    return accA, accB, denA, denB


def kernel(x, edge_index, W1, a_src1, a_dst1, b1, W2, a_src2, a_dst2, b2):
    pad_ids = (N + (jnp.arange(EP - E, dtype=jnp.int32) % (NP - N)))
    srcP = jnp.concatenate([edge_index[0], pad_ids]).reshape(EP // CH, CH)
    dstP = jnp.concatenate([edge_index[1], pad_ids]).reshape(EP // CH, CH)
    xP = jnp.concatenate([x, jnp.zeros((NP - N, D), jnp.float32)])
    z128 = jnp.zeros((NP, D), jnp.float32)
    z1 = jnp.zeros((NP,), jnp.float32)

    asv1 = a_src1.reshape(1, D)
    adv1 = a_dst1.reshape(1, D)
    asv2 = a_src2.reshape(1, D)
    adv2 = a_dst2.reshape(1, D)
    b1r = b1.reshape(1, D)
    b2r = b2.reshape(1, D)

    h1, asP1, adP1, am1 = _tc_pre(xP, W1, asv1, adv1)
    accA1, accB1, denA1, denB1 = _sc_edge(h1, asP1, adP1, am1, srcP, dstP,
                                          z128, z1)
    h2, asP2, adP2, am2 = _tc_mid(accA1, accB1, denA1.reshape(GRID * 8, 128),
                                  denB1.reshape(GRID * 8, 128), h1, asP1,
                                  adP1, am1, b1r, W2, asv2, adv2)
    accA2, accB2, denA2, denB2 = _sc_edge(h2, asP2, adP2, am2, srcP, dstP,
                                          z128, z1)
    return _tc_post(accA2, accB2, denA2.reshape(GRID * 8, 128),
                    denB2.reshape(GRID * 8, 128), h2, asP2, adP2, am2, b2r)


# speedup vs baseline: 1.2105x; 1.2105x over previous
import jax
import jax.numpy as jnp
from jax import lax
from jax.experimental import pallas as pl
from jax.experimental.pallas import tpu as pltpu
from jax.experimental.pallas import tpu_sc as plsc

N = 10000
E = 320000
D = 128
NEG_SLOPE = 0.2

NP = 10240
EP = 327680
BLK = 1024
GRID = NP // BLK
NCHUNK = 80
CH = 128
EROWS = NCHUNK
SP_ROWS = NP // 16


def _lrelu(t):
    return jnp.maximum(t, NEG_SLOPE * t)


_GDN = jax.lax.GatherDimensionNumbers(
    offset_dims=(), collapsed_slice_dims=(0,), start_index_map=(0,))


def _splat_lane(v16, k):
    idx = jnp.full((16,), k, jnp.int32)
    return jax.lax.gather(
        v16, idx[:, None], _GDN, (1,),
        mode=jax.lax.GatherScatterMode.PROMISE_IN_BOUNDS)


def _attn_epilogue(h_blk, asv, adv, asP_ref, adP_ref, am_ref, smax_ref, i):
    as_b = jnp.sum(h_blk * asv, axis=1)
    ad_b = jnp.sum(h_blk * adv, axis=1)
    asP_ref[...] = as_b.reshape(8, 128)
    adP_ref[...] = ad_b.reshape(8, 128)
    m = jnp.max(as_b)

    @pl.when(i == 0)
    def _():
        smax_ref[0, 0] = m

    @pl.when(i > 0)
    def _():
        smax_ref[0, 0] = jnp.maximum(smax_ref[0, 0], m)

    @pl.when(i == GRID - 1)
    def _():
        am_ref[...] = jnp.full((8, 128), smax_ref[0, 0], jnp.float32)


def _tc_pre_body(x_ref, w_ref, asv_ref, adv_ref,
                 h_ref, asP_ref, adP_ref, am_ref, smax_ref):
    i = pl.program_id(0)
    h_blk = jnp.dot(x_ref[...], w_ref[...], preferred_element_type=jnp.float32)
    h_ref[...] = h_blk
    _attn_epilogue(h_blk, asv_ref[...], adv_ref[...], asP_ref, adP_ref,
                   am_ref, smax_ref, i)


def _tc_pre(x, w, asv, adv):
    return pl.pallas_call(
        _tc_pre_body,
        grid=(GRID,),
        in_specs=[
            pl.BlockSpec((BLK, D), lambda i: (i, 0)),
            pl.BlockSpec((D, D), lambda i: (0, 0)),
            pl.BlockSpec((1, D), lambda i: (0, 0)),
            pl.BlockSpec((1, D), lambda i: (0, 0)),
        ],
        out_specs=[
            pl.BlockSpec((BLK, D), lambda i: (i, 0)),
            pl.BlockSpec((8, 128), lambda i: (i, 0)),
            pl.BlockSpec((8, 128), lambda i: (i, 0)),
            pl.BlockSpec((8, 128), lambda i: (0, 0)),
        ],
        out_shape=[
            jax.ShapeDtypeStruct((NP, D), jnp.float32),
            jax.ShapeDtypeStruct((GRID * 8, 128), jnp.float32),
            jax.ShapeDtypeStruct((GRID * 8, 128), jnp.float32),
            jax.ShapeDtypeStruct((8, 128), jnp.float32),
        ],
        scratch_shapes=[pltpu.SMEM((1, 1), jnp.float32)],
    )(x, w, asv, adv)


def _combine(accA_ref, accB_ref, denA_ref, denB_ref, h_ref, asP_ref, adP_ref,
             am_ref, b_ref):
    as_b = asP_ref[...].reshape(BLK)
    ad_b = adP_ref[...].reshape(BLK)
    am = am_ref[0, 0]
    exs = jnp.exp(_lrelu(as_b + ad_b) - _lrelu(am + ad_b))
    num = accA_ref[...] + accB_ref[...] + exs[:, None] * h_ref[...]
    den = denA_ref[...].reshape(BLK) + denB_ref[...].reshape(BLK) + exs
    return num / (den + 1e-16)[:, None] + b_ref[...]


_SPEC_ROWS = pl.BlockSpec((BLK, D), lambda i: (i, 0))
_SPEC_8x128 = pl.BlockSpec((8, 128), lambda i: (i, 0))
_SPEC_CONST = pl.BlockSpec((8, 128), lambda i: (0, 0))
_SPEC_VEC = pl.BlockSpec((1, D), lambda i: (0, 0))


def _tc_mid_body(accA_ref, accB_ref, denA_ref, denB_ref, h_ref, asP_ref,
                 adP_ref, am_ref, b_ref, w_ref, asv_ref, adv_ref,
                 h2_ref, asP2_ref, adP2_ref, am2_ref, smax_ref):
    i = pl.program_id(0)
    feats = _combine(accA_ref, accB_ref, denA_ref, denB_ref, h_ref, asP_ref,
                     adP_ref, am_ref, b_ref)
    h2_blk = jnp.dot(feats, w_ref[...], preferred_element_type=jnp.float32)
    h2_ref[...] = h2_blk
    _attn_epilogue(h2_blk, asv_ref[...], adv_ref[...], asP2_ref, adP2_ref,
                   am2_ref, smax_ref, i)


def _tc_mid(accA, accB, denA, denB, h, asP, adP, am, b, w, asv, adv):
    return pl.pallas_call(
        _tc_mid_body,
        grid=(GRID,),
        in_specs=[
            _SPEC_ROWS, _SPEC_ROWS, _SPEC_8x128, _SPEC_8x128, _SPEC_ROWS,
            _SPEC_8x128, _SPEC_8x128, _SPEC_CONST, _SPEC_VEC,
            pl.BlockSpec((D, D), lambda i: (0, 0)), _SPEC_VEC, _SPEC_VEC,
        ],
        out_specs=[
            _SPEC_ROWS, _SPEC_8x128, _SPEC_8x128, _SPEC_CONST,
        ],
        out_shape=[
            jax.ShapeDtypeStruct((NP, D), jnp.float32),
            jax.ShapeDtypeStruct((GRID * 8, 128), jnp.float32),
            jax.ShapeDtypeStruct((GRID * 8, 128), jnp.float32),
            jax.ShapeDtypeStruct((8, 128), jnp.float32),
        ],
        scratch_shapes=[pltpu.SMEM((1, 1), jnp.float32)],
    )(accA, accB, denA, denB, h, asP, adP, am, b, w, asv, adv)


def _tc_post_body(accA_ref, accB_ref, denA_ref, denB_ref, h_ref, asP_ref,
                  adP_ref, am_ref, b_ref, out_ref):
    out_ref[...] = _combine(accA_ref, accB_ref, denA_ref, denB_ref, h_ref,
                            asP_ref, adP_ref, am_ref, b_ref)


def _tc_post(accA, accB, denA, denB, h, asP, adP, am, b):
    return pl.pallas_call(
        _tc_post_body,
        grid=(GRID,),
        in_specs=[
            _SPEC_ROWS, _SPEC_ROWS, _SPEC_8x128, _SPEC_8x128, _SPEC_ROWS,
            _SPEC_8x128, _SPEC_8x128, _SPEC_CONST, _SPEC_VEC,
        ],
        out_specs=pl.BlockSpec((BLK, D), lambda i: (i, 0)),
        out_shape=jax.ShapeDtypeStruct((N, D), jnp.float32),
    )(accA, accB, denA, denB, h, asP, adP, am, b)


NSUP = NCHUNK // 8


def _sc_alpha_body(asP_hbm, adP_hbm, am_hbm, src_hbm, dst_hbm, z1_hbm,
                   ex_hbm, denA_hbm, denB_hbm,
                   as_v, ad_v, am_v, src8_v, dst8_v, ex8_v, dsem,
                   den_sp):
    cid = lax.axis_index("c")
    sid = lax.axis_index("s")
    w = cid * 16 + sid

    pltpu.sync_copy(asP_hbm, as_v)
    pltpu.sync_copy(adP_hbm, ad_v)
    pltpu.sync_copy(am_hbm.at[pl.ds(0, 1)], am_v)

    @pl.when(sid == 0)
    def _():
        pltpu.sync_copy(z1_hbm, den_sp)

    plsc.subcore_barrier()

    amax16 = am_v[0, pl.ds(0, 16)]

    def superstep(g, carry):
        br = pl.ds(w * EROWS + g * 8, 8)
        pltpu.sync_copy(src_hbm.at[br], src8_v)
        pltpu.sync_copy(dst_hbm.at[br], dst8_v)

        waits = []
        for c8 in range(8):
            for j in range(8):
                sl = pl.ds(16 * j, 16)
                s16 = src8_v[c8, sl]
                d16 = dst8_v[c8, sl]
                a_s = plsc.load_gather(as_v, [s16 >> 7, s16 & 127])
                a_d = plsc.load_gather(ad_v, [d16 >> 7, d16 & 127])
                al = _lrelu(a_s + a_d)
                ub = _lrelu(amax16 + a_d)
                ex8_v[c8, sl] = jnp.exp(al - ub)
            waits.append(pltpu.async_copy(
                ex8_v.at[c8], den_sp.at[dst8_v.at[c8]], dsem, add=True))
        pltpu.sync_copy(ex8_v, ex_hbm.at[br])
        for h in waits:
            h.wait()
        return carry

    lax.fori_loop(0, NSUP, superstep, 0)
    plsc.subcore_barrier()

    @pl.when(jnp.logical_and(cid == 0, sid == 0))
    def _():
        pltpu.sync_copy(den_sp, denA_hbm)

    @pl.when(jnp.logical_and(cid == 1, sid == 0))
    def _():
        pltpu.sync_copy(den_sp, denB_hbm)


def _sc_rows_body(h_hbm, ex_hbm, src_hbm, dst_hbm, z128_hbm,
                  accA_hbm, accB_hbm,
                  src8_v, dst8_v, ex8_v, rows_a, rows_b, sem_a, sem_b,
                  ssem_a, ssem_b, out_sp):
    cid = lax.axis_index("c")
    sid = lax.axis_index("s")
    w = cid * 16 + sid

    rz = pl.ds(sid * SP_ROWS, SP_ROWS)
    pltpu.sync_copy(z128_hbm.at[rz], out_sp.at[rz])
    plsc.subcore_barrier()

    def stage(batch, half):
        br = pl.ds(w * EROWS + batch * 8, 8)
        hf = pl.ds(half * 8, 8)
        pltpu.sync_copy(src_hbm.at[br], src8_v.at[hf])
        pltpu.sync_copy(dst_hbm.at[br], dst8_v.at[hf])
        pltpu.sync_copy(ex_hbm.at[br], ex8_v.at[hf])

    stage(0, 0)
    pltpu.async_copy(h_hbm.at[src8_v.at[0]], rows_a, sem_a)

    def superstep(g, carry):
        gp = g & 1

        @pl.when(g < NSUP - 1)
        def _():
            stage(g + 1, 1 - gp)

        for c8 in range(8):
            rows, sem = (rows_a, sem_a) if c8 % 2 == 0 else (rows_b, sem_b)
            orows, osem = (rows_b, sem_b) if c8 % 2 == 0 else (rows_a, sem_a)
            ossem = ssem_b if c8 % 2 == 0 else ssem_a
            myssem = ssem_a if c8 % 2 == 0 else ssem_b
            r = gp * 8 + c8

            pltpu.make_async_copy(
                h_hbm.at[src8_v.at[r]], rows, sem).wait()

            def _wait_prev():
                pltpu.make_async_copy(
                    orows, out_sp.at[dst8_v.at[r]], ossem).wait()

            if c8 == 0:
                @pl.when(g > 0)
                def _():
                    _wait_prev()
            else:
                _wait_prev()

            if c8 < 7:
                pltpu.async_copy(h_hbm.at[src8_v.at[r + 1]], orows, osem)
            else:
                @pl.when(g < NSUP - 1)
                def _():
                    pltpu.async_copy(
                        h_hbm.at[src8_v.at[(1 - gp) * 8]], orows, osem)

            for grp in range(8):
                e16 = ex8_v[r, pl.ds(16 * grp, 16)]

                def scalek(q, cc, e16=e16, grp=grp, rows=rows):
                    for u in range(4):
                        k2 = q * 4 + u
                        e = _splat_lane(e16, k2)
                        for j in range(8):
                            sl = pl.ds(16 * j, 16)
                            rr = 16 * grp + k2
                            rows[rr, sl] = rows[rr, sl] * e
                    return cc

                lax.fori_loop(0, 4, scalek, 0)

            pltpu.async_copy(rows, out_sp.at[dst8_v.at[r]], myssem, add=True)
        return carry

    lax.fori_loop(0, NSUP, superstep, 0)
    pltpu.make_async_copy(
        rows_b, out_sp.at[dst8_v.at[15]], ssem_b).wait()
    plsc.subcore_barrier()

    @pl.when(cid == 0)
    def _():
        pltpu.sync_copy(out_sp.at[rz], accA_hbm.at[rz])

    @pl.when(cid == 1)
    def _():
        pltpu.sync_copy(out_sp.at[rz], accB_hbm.at[rz])


def _sc_edge(h, asP, adP, am, srcP, dstP, z128, z1):
    mesh = plsc.VectorSubcoreMesh(core_axis_name="c", subcore_axis_name="s")
    alpha = pl.kernel(
        _sc_alpha_body,
        mesh=mesh,
        compiler_params=pltpu.CompilerParams(needs_layout_passes=False),
        out_type=[
            jax.ShapeDtypeStruct((EP // CH, CH), jnp.float32),
            jax.ShapeDtypeStruct((NP,), jnp.float32),
            jax.ShapeDtypeStruct((NP,), jnp.float32),
        ],
        scratch_types=[
            pltpu.VMEM((GRID * 8, 128), jnp.float32),
            pltpu.VMEM((GRID * 8, 128), jnp.float32),
            pltpu.VMEM((1, 128), jnp.float32),
            pltpu.VMEM((8, CH), jnp.int32),
            pltpu.VMEM((8, CH), jnp.int32),
            pltpu.VMEM((8, CH), jnp.float32),
            pltpu.SemaphoreType.DMA,
            pltpu.VMEM_SHARED((NP,), jnp.float32),
        ],
    )
    exP, denA, denB = alpha(asP, adP, am, srcP, dstP, z1)

    rows = pl.kernel(
        _sc_rows_body,
        mesh=mesh,
        compiler_params=pltpu.CompilerParams(needs_layout_passes=False),
        out_type=[
            jax.ShapeDtypeStruct((NP, D), jnp.float32),
            jax.ShapeDtypeStruct((NP, D), jnp.float32),
        ],
        scratch_types=[
            pltpu.VMEM((16, CH), jnp.int32),
            pltpu.VMEM((16, CH), jnp.int32),
            pltpu.VMEM((16, CH), jnp.float32),
            pltpu.VMEM((CH, D), jnp.float32),
            pltpu.VMEM((CH, D), jnp.float32),
            pltpu.SemaphoreType.DMA,
            pltpu.SemaphoreType.DMA,
            pltpu.SemaphoreType.DMA,
            pltpu.SemaphoreType.DMA,
            pltpu.VMEM_SHARED((NP, D), jnp.float32),
        ],
    )
    accA, accB = rows(h, exP, srcP, dstP, z128)
    return accA, accB, denA, denB


def kernel(x, edge_index, W1, a_src1, a_dst1, b1, W2, a_src2, a_dst2, b2):
    pad_ids = (N + (jnp.arange(EP - E, dtype=jnp.int32) % (NP - N)))
    srcP = jnp.concatenate([edge_index[0], pad_ids]).reshape(EP // CH, CH)
    dstP = jnp.concatenate([edge_index[1], pad_ids]).reshape(EP // CH, CH)
    xP = jnp.concatenate([x, jnp.zeros((NP - N, D), jnp.float32)])
    z128 = jnp.zeros((NP, D), jnp.float32)
    z1 = jnp.zeros((NP,), jnp.float32)

    asv1 = a_src1.reshape(1, D)
    adv1 = a_dst1.reshape(1, D)
    asv2 = a_src2.reshape(1, D)
    adv2 = a_dst2.reshape(1, D)
    b1r = b1.reshape(1, D)
    b2r = b2.reshape(1, D)

    h1, asP1, adP1, am1 = _tc_pre(xP, W1, asv1, adv1)
    accA1, accB1, denA1, denB1 = _sc_edge(h1, asP1, adP1, am1, srcP, dstP,
                                          z128, z1)
    h2, asP2, adP2, am2 = _tc_mid(accA1, accB1, denA1.reshape(GRID * 8, 128),
                                  denB1.reshape(GRID * 8, 128), h1, asP1,
                                  adP1, am1, b1r, W2, asv2, adv2)
    accA2, accB2, denA2, denB2 = _sc_edge(h2, asP2, adP2, am2, srcP, dstP,
                                          z128, z1)
    return _tc_post(accA2, accB2, denA2.reshape(GRID * 8, 128),
                    denB2.reshape(GRID * 8, 128), h2, asP2, adP2, am2, b2r)


# speedup vs baseline: 1.4375x; 1.1875x over previous
import jax
import jax.numpy as jnp
from jax import lax
from jax.experimental import pallas as pl
from jax.experimental.pallas import tpu as pltpu
from jax.experimental.pallas import tpu_sc as plsc

N = 10000
E = 320000
D = 128
NEG_SLOPE = 0.2

NP = 10240
EP = 327680
BLK = 1024
GRID = NP // BLK
NCHUNK = 80
CH = 128
EROWS = NCHUNK
SP_ROWS = NP // 16


def _lrelu(t):
    return jnp.maximum(t, NEG_SLOPE * t)


_GDN = jax.lax.GatherDimensionNumbers(
    offset_dims=(), collapsed_slice_dims=(0,), start_index_map=(0,))


def _splat_lane(v16, k):
    idx = jnp.full((16,), k, jnp.int32)
    return jax.lax.gather(
        v16, idx[:, None], _GDN, (1,),
        mode=jax.lax.GatherScatterMode.PROMISE_IN_BOUNDS)


def _attn_epilogue(h_blk, asv, adv, asP_ref, adP_ref, am_ref, smax_ref, i):
    as_b = jnp.sum(h_blk * asv, axis=1)
    ad_b = jnp.sum(h_blk * adv, axis=1)
    asP_ref[...] = as_b.reshape(8, 128)
    adP_ref[...] = ad_b.reshape(8, 128)
    m = jnp.max(as_b)

    @pl.when(i == 0)
    def _():
        smax_ref[0, 0] = m

    @pl.when(i > 0)
    def _():
        smax_ref[0, 0] = jnp.maximum(smax_ref[0, 0], m)

    @pl.when(i == GRID - 1)
    def _():
        am_ref[...] = jnp.full((8, 128), smax_ref[0, 0], jnp.float32)


def _tc_pre_body(x_ref, w_ref, asv_ref, adv_ref,
                 h_ref, asP_ref, adP_ref, am_ref, smax_ref):
    i = pl.program_id(0)
    h_blk = jnp.dot(x_ref[...], w_ref[...], preferred_element_type=jnp.float32)
    h_ref[...] = h_blk
    _attn_epilogue(h_blk, asv_ref[...], adv_ref[...], asP_ref, adP_ref,
                   am_ref, smax_ref, i)


def _tc_pre(x, w, asv, adv):
    return pl.pallas_call(
        _tc_pre_body,
        grid=(GRID,),
        in_specs=[
            pl.BlockSpec((BLK, D), lambda i: (i, 0)),
            pl.BlockSpec((D, D), lambda i: (0, 0)),
            pl.BlockSpec((1, D), lambda i: (0, 0)),
            pl.BlockSpec((1, D), lambda i: (0, 0)),
        ],
        out_specs=[
            pl.BlockSpec((BLK, D), lambda i: (i, 0)),
            pl.BlockSpec((8, 128), lambda i: (i, 0)),
            pl.BlockSpec((8, 128), lambda i: (i, 0)),
            pl.BlockSpec((8, 128), lambda i: (0, 0)),
        ],
        out_shape=[
            jax.ShapeDtypeStruct((NP, D), jnp.float32),
            jax.ShapeDtypeStruct((GRID * 8, 128), jnp.float32),
            jax.ShapeDtypeStruct((GRID * 8, 128), jnp.float32),
            jax.ShapeDtypeStruct((8, 128), jnp.float32),
        ],
        scratch_shapes=[pltpu.SMEM((1, 1), jnp.float32)],
    )(x, w, asv, adv)


def _combine(accA_ref, accB_ref, denA_ref, denB_ref, h_ref, asP_ref, adP_ref,
             am_ref, b_ref):
    as_b = asP_ref[...].reshape(BLK)
    ad_b = adP_ref[...].reshape(BLK)
    am = am_ref[0, 0]
    exs = jnp.exp(_lrelu(as_b + ad_b) - _lrelu(am + ad_b))
    num = accA_ref[...] + accB_ref[...] + exs[:, None] * h_ref[...]
    den = denA_ref[...].reshape(BLK) + denB_ref[...].reshape(BLK) + exs
    return num / (den + 1e-16)[:, None] + b_ref[...]


_SPEC_ROWS = pl.BlockSpec((BLK, D), lambda i: (i, 0))
_SPEC_8x128 = pl.BlockSpec((8, 128), lambda i: (i, 0))
_SPEC_CONST = pl.BlockSpec((8, 128), lambda i: (0, 0))
_SPEC_VEC = pl.BlockSpec((1, D), lambda i: (0, 0))


def _tc_mid_body(accA_ref, accB_ref, denA_ref, denB_ref, h_ref, asP_ref,
                 adP_ref, am_ref, b_ref, w_ref, asv_ref, adv_ref,
                 h2_ref, asP2_ref, adP2_ref, am2_ref, smax_ref):
    i = pl.program_id(0)
    feats = _combine(accA_ref, accB_ref, denA_ref, denB_ref, h_ref, asP_ref,
                     adP_ref, am_ref, b_ref)
    h2_blk = jnp.dot(feats, w_ref[...], preferred_element_type=jnp.float32)
    h2_ref[...] = h2_blk
    _attn_epilogue(h2_blk, asv_ref[...], adv_ref[...], asP2_ref, adP2_ref,
                   am2_ref, smax_ref, i)


def _tc_mid(accA, accB, denA, denB, h, asP, adP, am, b, w, asv, adv):
    return pl.pallas_call(
        _tc_mid_body,
        grid=(GRID,),
        in_specs=[
            _SPEC_ROWS, _SPEC_ROWS, _SPEC_8x128, _SPEC_8x128, _SPEC_ROWS,
            _SPEC_8x128, _SPEC_8x128, _SPEC_CONST, _SPEC_VEC,
            pl.BlockSpec((D, D), lambda i: (0, 0)), _SPEC_VEC, _SPEC_VEC,
        ],
        out_specs=[
            _SPEC_ROWS, _SPEC_8x128, _SPEC_8x128, _SPEC_CONST,
        ],
        out_shape=[
            jax.ShapeDtypeStruct((NP, D), jnp.float32),
            jax.ShapeDtypeStruct((GRID * 8, 128), jnp.float32),
            jax.ShapeDtypeStruct((GRID * 8, 128), jnp.float32),
            jax.ShapeDtypeStruct((8, 128), jnp.float32),
        ],
        scratch_shapes=[pltpu.SMEM((1, 1), jnp.float32)],
    )(accA, accB, denA, denB, h, asP, adP, am, b, w, asv, adv)


def _tc_post_body(accA_ref, accB_ref, denA_ref, denB_ref, h_ref, asP_ref,
                  adP_ref, am_ref, b_ref, out_ref):
    out_ref[...] = _combine(accA_ref, accB_ref, denA_ref, denB_ref, h_ref,
                            asP_ref, adP_ref, am_ref, b_ref)


def _tc_post(accA, accB, denA, denB, h, asP, adP, am, b):
    return pl.pallas_call(
        _tc_post_body,
        grid=(GRID,),
        in_specs=[
            _SPEC_ROWS, _SPEC_ROWS, _SPEC_8x128, _SPEC_8x128, _SPEC_ROWS,
            _SPEC_8x128, _SPEC_8x128, _SPEC_CONST, _SPEC_VEC,
        ],
        out_specs=pl.BlockSpec((BLK, D), lambda i: (i, 0)),
        out_shape=jax.ShapeDtypeStruct((N, D), jnp.float32),
    )(accA, accB, denA, denB, h, asP, adP, am, b)


NSUP = NCHUNK // 8


def _sc_alpha_body(asP_hbm, adP_hbm, am_hbm, src_hbm, dst_hbm, z1_hbm,
                   ex_hbm, denA_hbm, denB_hbm,
                   as_v, ad_v, am_v, src8_v, dst8_v, ex8_v, dsem, bsem, wsem,
                   den_sp):
    cid = lax.axis_index("c")
    sid = lax.axis_index("s")
    w = cid * 16 + sid

    def brow(batch):
        return pl.ds(w * EROWS + batch * 8, 8)

    def half(p):
        return pl.ds(p * 8, 8)

    pltpu.sync_copy(asP_hbm, as_v)
    pltpu.sync_copy(adP_hbm, ad_v)
    pltpu.sync_copy(am_hbm.at[pl.ds(0, 1)], am_v)
    pltpu.sync_copy(src_hbm.at[brow(0)], src8_v.at[half(0)])
    pltpu.sync_copy(dst_hbm.at[brow(0)], dst8_v.at[half(0)])

    @pl.when(sid == 0)
    def _():
        pltpu.sync_copy(z1_hbm, den_sp)

    plsc.subcore_barrier()

    amax16 = am_v[0, pl.ds(0, 16)]

    def superstep(g, carry):
        gp = g & 1

        @pl.when(g > 0)
        def _():
            pltpu.make_async_copy(
                src_hbm.at[brow(g)], src8_v.at[half(gp)], bsem).wait()
            pltpu.make_async_copy(
                dst_hbm.at[brow(g)], dst8_v.at[half(gp)], bsem).wait()
            pltpu.make_async_copy(
                ex8_v.at[half(1 - gp)], ex_hbm.at[brow(g)], wsem).wait()
            for c8 in range(8):
                pltpu.make_async_copy(
                    ex8_v.at[c8], den_sp.at[dst8_v.at[c8]], dsem).wait()

        @pl.when(g < NSUP - 1)
        def _():
            pltpu.async_copy(
                src_hbm.at[brow(g + 1)], src8_v.at[half(1 - gp)], bsem)
            pltpu.async_copy(
                dst_hbm.at[brow(g + 1)], dst8_v.at[half(1 - gp)], bsem)

        for c8 in range(8):
            r = gp * 8 + c8
            for j in range(8):
                sl = pl.ds(16 * j, 16)
                s16 = src8_v[r, sl]
                d16 = dst8_v[r, sl]
                a_s = plsc.load_gather(as_v, [s16 >> 7, s16 & 127])
                a_d = plsc.load_gather(ad_v, [d16 >> 7, d16 & 127])
                al = _lrelu(a_s + a_d)
                ub = _lrelu(amax16 + a_d)
                ex8_v[r, sl] = jnp.exp(al - ub)
            pltpu.async_copy(
                ex8_v.at[r], den_sp.at[dst8_v.at[r]], dsem, add=True)
        pltpu.async_copy(ex8_v.at[half(gp)], ex_hbm.at[brow(g)], wsem)
        return carry

    lax.fori_loop(0, NSUP, superstep, 0)

    pltpu.make_async_copy(
        ex8_v.at[half(1)], ex_hbm.at[brow(NSUP - 1)], wsem).wait()
    for c8 in range(8):
        pltpu.make_async_copy(
            ex8_v.at[8 + c8], den_sp.at[dst8_v.at[8 + c8]], dsem).wait()
    plsc.subcore_barrier()

    @pl.when(jnp.logical_and(cid == 0, sid == 0))
    def _():
        pltpu.sync_copy(den_sp, denA_hbm)

    @pl.when(jnp.logical_and(cid == 1, sid == 0))
    def _():
        pltpu.sync_copy(den_sp, denB_hbm)


def _sc_rows_body(h_hbm, ex_hbm, src_hbm, dst_hbm, z128_hbm,
                  accA_hbm, accB_hbm,
                  src8_v, dst8_v, ex8_v, rows_a, rows_b, sem_a, sem_b,
                  ssem_a, ssem_b, bsem, out_sp):
    cid = lax.axis_index("c")
    sid = lax.axis_index("s")
    w = cid * 16 + sid

    rz = pl.ds(sid * SP_ROWS, SP_ROWS)
    pltpu.sync_copy(z128_hbm.at[rz], out_sp.at[rz])
    plsc.subcore_barrier()

    def stage(batch, half, copy):
        br = pl.ds(w * EROWS + batch * 8, 8)
        hf = pl.ds(half * 8, 8)
        copy(src_hbm.at[br], src8_v.at[hf], bsem)
        copy(dst_hbm.at[br], dst8_v.at[hf], bsem)
        copy(ex_hbm.at[br], ex8_v.at[hf], bsem)

    def _sync3(src, dst, sem):
        pltpu.sync_copy(src, dst)

    def _fire3(src, dst, sem):
        pltpu.async_copy(src, dst, sem)

    def _drain3(src, dst, sem):
        pltpu.make_async_copy(src, dst, sem).wait()

    stage(0, 0, _sync3)
    pltpu.async_copy(h_hbm.at[src8_v.at[0]], rows_a, sem_a)

    def superstep(g, carry):
        gp = g & 1

        @pl.when(g < NSUP - 1)
        def _():
            stage(g + 1, 1 - gp, _fire3)

        for c8 in range(8):
            rows, sem = (rows_a, sem_a) if c8 % 2 == 0 else (rows_b, sem_b)
            orows, osem = (rows_b, sem_b) if c8 % 2 == 0 else (rows_a, sem_a)
            ossem = ssem_b if c8 % 2 == 0 else ssem_a
            myssem = ssem_a if c8 % 2 == 0 else ssem_b
            r = gp * 8 + c8

            pltpu.make_async_copy(
                h_hbm.at[src8_v.at[r]], rows, sem).wait()

            def _wait_prev():
                pltpu.make_async_copy(
                    orows, out_sp.at[dst8_v.at[r]], ossem).wait()

            if c8 == 0:
                @pl.when(g > 0)
                def _():
                    _wait_prev()
            else:
                _wait_prev()

            if c8 < 7:
                pltpu.async_copy(h_hbm.at[src8_v.at[r + 1]], orows, osem)
            else:
                @pl.when(g < NSUP - 1)
                def _():
                    stage(g + 1, 1 - gp, _drain3)
                    pltpu.async_copy(
                        h_hbm.at[src8_v.at[(1 - gp) * 8]], orows, osem)

            for grp in range(8):
                e16 = ex8_v[r, pl.ds(16 * grp, 16)]

                def scalek(k2, cc, e16=e16, grp=grp, rows=rows):
                    e = _splat_lane(e16, k2)
                    for j in range(8):
                        sl = pl.ds(16 * j, 16)
                        rows[16 * grp + k2, sl] = rows[16 * grp + k2, sl] * e
                    return cc

                lax.fori_loop(0, 16, scalek, 0)

            pltpu.async_copy(rows, out_sp.at[dst8_v.at[r]], myssem, add=True)
        return carry

    lax.fori_loop(0, NSUP, superstep, 0)
    pltpu.make_async_copy(
        rows_b, out_sp.at[dst8_v.at[15]], ssem_b).wait()
    plsc.subcore_barrier()

    @pl.when(cid == 0)
    def _():
        pltpu.sync_copy(out_sp.at[rz], accA_hbm.at[rz])

    @pl.when(cid == 1)
    def _():
        pltpu.sync_copy(out_sp.at[rz], accB_hbm.at[rz])


def _sc_edge(h, asP, adP, am, srcP, dstP, z128, z1):
    mesh = plsc.VectorSubcoreMesh(core_axis_name="c", subcore_axis_name="s")
    alpha = pl.kernel(
        _sc_alpha_body,
        mesh=mesh,
        compiler_params=pltpu.CompilerParams(needs_layout_passes=False),
        out_type=[
            jax.ShapeDtypeStruct((EP // CH, CH), jnp.float32),
            jax.ShapeDtypeStruct((NP,), jnp.float32),
            jax.ShapeDtypeStruct((NP,), jnp.float32),
        ],
        scratch_types=[
            pltpu.VMEM((GRID * 8, 128), jnp.float32),
            pltpu.VMEM((GRID * 8, 128), jnp.float32),
            pltpu.VMEM((1, 128), jnp.float32),
            pltpu.VMEM((16, CH), jnp.int32),
            pltpu.VMEM((16, CH), jnp.int32),
            pltpu.VMEM((16, CH), jnp.float32),
            pltpu.SemaphoreType.DMA,
            pltpu.SemaphoreType.DMA,
            pltpu.SemaphoreType.DMA,
            pltpu.VMEM_SHARED((NP,), jnp.float32),
        ],
    )
    exP, denA, denB = alpha(asP, adP, am, srcP, dstP, z1)

    rows = pl.kernel(
        _sc_rows_body,
        mesh=mesh,
        compiler_params=pltpu.CompilerParams(needs_layout_passes=False),
        out_type=[
            jax.ShapeDtypeStruct((NP, D), jnp.float32),
            jax.ShapeDtypeStruct((NP, D), jnp.float32),
        ],
        scratch_types=[
            pltpu.VMEM((16, CH), jnp.int32),
            pltpu.VMEM((16, CH), jnp.int32),
            pltpu.VMEM((16, CH), jnp.float32),
            pltpu.VMEM((CH, D), jnp.float32),
            pltpu.VMEM((CH, D), jnp.float32),
            pltpu.SemaphoreType.DMA,
            pltpu.SemaphoreType.DMA,
            pltpu.SemaphoreType.DMA,
            pltpu.SemaphoreType.DMA,
            pltpu.SemaphoreType.DMA,
            pltpu.VMEM_SHARED((NP, D), jnp.float32),
        ],
    )
    accA, accB = rows(h, exP, srcP, dstP, z128)
    return accA, accB, denA, denB


def kernel(x, edge_index, W1, a_src1, a_dst1, b1, W2, a_src2, a_dst2, b2):
    pad_ids = (N + (jnp.arange(EP - E, dtype=jnp.int32) % (NP - N)))
    srcP = jnp.concatenate([edge_index[0], pad_ids]).reshape(EP // CH, CH)
    dstP = jnp.concatenate([edge_index[1], pad_ids]).reshape(EP // CH, CH)
    xP = jnp.concatenate([x, jnp.zeros((NP - N, D), jnp.float32)])
    z128 = jnp.zeros((NP, D), jnp.float32)
    z1 = jnp.zeros((NP,), jnp.float32)

    asv1 = a_src1.reshape(1, D)
    adv1 = a_dst1.reshape(1, D)
    asv2 = a_src2.reshape(1, D)
    adv2 = a_dst2.reshape(1, D)
    b1r = b1.reshape(1, D)
    b2r = b2.reshape(1, D)

    h1, asP1, adP1, am1 = _tc_pre(xP, W1, asv1, adv1)
    accA1, accB1, denA1, denB1 = _sc_edge(h1, asP1, adP1, am1, srcP, dstP,
                                          z128, z1)
    h2, asP2, adP2, am2 = _tc_mid(accA1, accB1, denA1.reshape(GRID * 8, 128),
                                  denB1.reshape(GRID * 8, 128), h1, asP1,
                                  adP1, am1, b1r, W2, asv2, adv2)
    accA2, accB2, denA2, denB2 = _sc_edge(h2, asP2, adP2, am2, srcP, dstP,
                                          z128, z1)
    return _tc_post(accA2, accB2, denA2.reshape(GRID * 8, 128),
                    denB2.reshape(GRID * 8, 128), h2, asP2, adP2, am2, b2r)
